# baseline (speedup 1.0000x reference)
import functools

import jax
import jax.numpy as jnp
from jax.experimental import pallas as pl
from jax.experimental.pallas import tpu as pltpu

NSTREAM = 5
RS = 80


def _s0_kernel(h_ref, w0_ref, out_ref):
    out_ref[...] = jnp.dot(
        h_ref[...].astype(jnp.bfloat16),
        w0_ref[...].astype(jnp.bfloat16),
        preferred_element_type=jnp.float32,
    ).astype(jnp.bfloat16)


def _fused_kernel(*refs, nm):
    a_refs = refs[:NSTREAM]
    s0_ref, b0_ref, w12_ref, b12_ref, out_ref, s12_ref = refs[NSTREAM:]
    t = pl.program_id(0)
    bm = NSTREAM * RS

    @pl.when(t < nm)
    def _phase1():
        for j, a_ref in enumerate(a_refs):
            acc = jnp.dot(
                a_ref[...].astype(jnp.bfloat16),
                s0_ref[...],
                preferred_element_type=jnp.float32,
            )
            hh = jnp.maximum(acc + b0_ref[...], 0.0).astype(jnp.bfloat16)
            s12_ref[pl.ds(t * bm + j * RS, RS), :] = jnp.dot(
                hh,
                w12_ref[...],
                preferred_element_type=jnp.float32,
            ).astype(jnp.bfloat16)

    @pl.when(t >= nm)
    def _phase2():
        for j, a_ref in enumerate(a_refs):
            acc = jnp.dot(
                a_ref[...].astype(jnp.bfloat16),
                s12_ref[...],
                preferred_element_type=jnp.float32,
            )
            out_ref[pl.ds(j * RS, RS), :] = acc + b12_ref[...]


@jax.jit
def kernel(H, A, W0, b0, W1, b1, W2, b2):
    n, nfeat = H.shape
    nhid = W0.shape[1]
    nclass = W1.shape[1]
    ndim = W2.shape[1]
    bm = NSTREAM * RS
    nm = n // bm

    s0 = pl.pallas_call(
        _s0_kernel,
        grid=(n // bm,),
        in_specs=[
            pl.BlockSpec((bm, nfeat), lambda i: (i, 0)),
            pl.BlockSpec((nfeat, nhid), lambda i: (0, 0)),
        ],
        out_specs=pl.BlockSpec((bm, nhid), lambda i: (i, 0)),
        out_shape=jax.ShapeDtypeStruct((n, nhid), jnp.bfloat16),
    )(H, W0)

    w12 = jnp.concatenate([W1, W2], axis=1).astype(jnp.bfloat16)
    b12 = jnp.concatenate([b1, b2])[None, :]
    b0_2d = b0[None, :]
    ncat = nclass + ndim

    full_spec = lambda shape: pl.BlockSpec(shape, lambda t: (0, 0))
    a_idx = lambda t: jnp.where(t < nm, t, t - nm)

    def stream_spec(j):
        return pl.BlockSpec(
            (RS, n), lambda t, j=j: (a_idx(t) * NSTREAM + j, 0))

    out = pl.pallas_call(
        functools.partial(_fused_kernel, nm=nm),
        grid=(2 * nm,),
        in_specs=[stream_spec(j) for j in range(NSTREAM)] + [
            full_spec((n, nhid)),
            full_spec((1, nhid)),
            full_spec((nhid, ncat)),
            full_spec((1, ncat)),
        ],
        out_specs=pl.BlockSpec((bm, ncat), lambda t: (jnp.maximum(t - nm, 0), 0)),
        out_shape=jax.ShapeDtypeStruct((n, ncat), jnp.float32),
        scratch_shapes=[pltpu.VMEM((n, ncat), jnp.bfloat16)],
        compiler_params=pltpu.CompilerParams(dimension_semantics=("arbitrary",)),
    )(*([A] * NSTREAM), s0, b0_2d, w12, b12)

    return (out[:, :nclass], out[:, nclass:])


# speedup vs baseline: 1.0126x; 1.0126x over previous
import functools

import jax
import jax.numpy as jnp
from jax.experimental import pallas as pl
from jax.experimental.pallas import tpu as pltpu

NSTREAM = 1
RS = 400


def _s0_kernel(h_ref, w0_ref, out_ref):
    out_ref[...] = jnp.dot(
        h_ref[...].astype(jnp.bfloat16),
        w0_ref[...].astype(jnp.bfloat16),
        preferred_element_type=jnp.float32,
    ).astype(jnp.bfloat16)


def _fused_kernel(*refs, nm):
    a_refs = refs[:NSTREAM]
    s0_ref, b0_ref, w12_ref, b12_ref, out_ref, s12_ref = refs[NSTREAM:]
    t = pl.program_id(0)
    bm = NSTREAM * RS

    @pl.when(t < nm)
    def _phase1():
        for j, a_ref in enumerate(a_refs):
            acc = jnp.dot(
                a_ref[...].astype(jnp.bfloat16),
                s0_ref[...],
                preferred_element_type=jnp.float32,
            )
            hh = jnp.maximum(acc + b0_ref[...], 0.0).astype(jnp.bfloat16)
            s12_ref[pl.ds(t * bm + j * RS, RS), :] = jnp.dot(
                hh,
                w12_ref[...],
                preferred_element_type=jnp.float32,
            ).astype(jnp.bfloat16)

    @pl.when(t >= nm)
    def _phase2():
        for j, a_ref in enumerate(a_refs):
            acc = jnp.dot(
                a_ref[...].astype(jnp.bfloat16),
                s12_ref[...],
                preferred_element_type=jnp.float32,
            )
            out_ref[pl.ds(j * RS, RS), :] = acc + b12_ref[...]


@jax.jit
def kernel(H, A, W0, b0, W1, b1, W2, b2):
    n, nfeat = H.shape
    nhid = W0.shape[1]
    nclass = W1.shape[1]
    ndim = W2.shape[1]
    bm = NSTREAM * RS
    nm = n // bm

    s0 = pl.pallas_call(
        _s0_kernel,
        grid=(n // bm,),
        in_specs=[
            pl.BlockSpec((bm, nfeat), lambda i: (i, 0)),
            pl.BlockSpec((nfeat, nhid), lambda i: (0, 0)),
        ],
        out_specs=pl.BlockSpec((bm, nhid), lambda i: (i, 0)),
        out_shape=jax.ShapeDtypeStruct((n, nhid), jnp.bfloat16),
    )(H, W0)

    w12 = jnp.concatenate([W1, W2], axis=1).astype(jnp.bfloat16)
    b12 = jnp.concatenate([b1, b2])[None, :]
    b0_2d = b0[None, :]
    ncat = nclass + ndim

    full_spec = lambda shape: pl.BlockSpec(shape, lambda t: (0, 0))
    a_idx = lambda t: jnp.where(t < nm, t, t - nm)

    def stream_spec(j):
        return pl.BlockSpec(
            (RS, n), lambda t, j=j: (a_idx(t) * NSTREAM + j, 0))

    out = pl.pallas_call(
        functools.partial(_fused_kernel, nm=nm),
        grid=(2 * nm,),
        in_specs=[stream_spec(j) for j in range(NSTREAM)] + [
            full_spec((n, nhid)),
            full_spec((1, nhid)),
            full_spec((nhid, ncat)),
            full_spec((1, ncat)),
        ],
        out_specs=pl.BlockSpec((bm, ncat), lambda t: (jnp.maximum(t - nm, 0), 0)),
        out_shape=jax.ShapeDtypeStruct((n, ncat), jnp.float32),
        scratch_shapes=[pltpu.VMEM((n, ncat), jnp.bfloat16)],
        compiler_params=pltpu.CompilerParams(dimension_semantics=("arbitrary",)),
    )(*([A] * NSTREAM), s0, b0_2d, w12, b12)

    return (out[:, :nclass], out[:, nclass:])


# speedup vs baseline: 1.0975x; 1.0839x over previous
import jax
import jax.numpy as jnp
import numpy as np
from jax import lax
from jax.experimental import pallas as pl
from jax.experimental.pallas import tpu as pltpu

N = 10000
B = 1664
NMC = 6
RE = N - NMC * B
NMR = NMC + 1
NM2 = NMR * NMC
NT = NM2 + NMC * (NMC + 1) // 2
SB = 2000


def _s0_kernel(h_ref, w0_ref, out_ref):
    out_ref[...] = jnp.dot(
        h_ref[...].astype(jnp.bfloat16),
        w0_ref[...].astype(jnp.bfloat16),
        preferred_element_type=jnp.float32,
    ).astype(jnp.bfloat16)


def _tri_kernel(i_tab, j_tab, a_hbm, s0_ref, s0e_ref, sliv_ref, b0_ref,
                w12_ref, b12_ref, out_ref, abuf, acc1, s12_ref, s12e_ref,
                oacc, sem):
    t = pl.program_id(0)
    i = i_tab[t]
    j = j_tab[t]
    slot = lax.rem(t, 2)

    def start_copy(u, s):
        iu = i_tab[u]
        ju = j_tab[u]
        r0 = iu * B
        c0 = ju * B

        @pl.when(iu < NMC)
        def _full():
            pltpu.make_async_copy(
                a_hbm.at[pl.ds(r0, B), pl.ds(c0, B)],
                abuf.at[s], sem.at[s]).start()

        @pl.when(iu == NMC)
        def _edge():
            pltpu.make_async_copy(
                a_hbm.at[pl.ds(r0, RE), pl.ds(c0, B)],
                abuf.at[s, pl.ds(0, RE)], sem.at[s]).start()

    def wait_copy(u, s):
        iu = i_tab[u]

        @pl.when(iu < NMC)
        def _full():
            pltpu.make_async_copy(
                a_hbm.at[pl.ds(0, B), pl.ds(0, B)],
                abuf.at[s], sem.at[s]).wait()

        @pl.when(iu == NMC)
        def _edge():
            pltpu.make_async_copy(
                a_hbm.at[pl.ds(0, RE), pl.ds(0, B)],
                abuf.at[s, pl.ds(0, RE)], sem.at[s]).wait()

    @pl.when(t == 0)
    def _prime():
        oacc[...] = jnp.zeros_like(oacc)
        start_copy(0, 0)

    @pl.when(t + 1 < NT)
    def _prefetch():
        start_copy(t + 1, lax.rem(t + 1, 2))

    wait_copy(t, slot)

    @pl.when(i < NMC)
    def _interior():
        at = abuf[slot].astype(jnp.bfloat16)
        sl = sliv_ref[pl.ds(i * B, B)]

        @pl.when(t < NM2)
        def _pass1():
            d = jnp.dot(at, s0_ref[j], preferred_element_type=jnp.float32)
            first = d + jnp.dot(sl, s0e_ref[...],
                                preferred_element_type=jnp.float32)
            acc1[...] = jnp.where(j == 0, first, acc1[...] + d)

            @pl.when(j == NMC - 1)
            def _finalize():
                hh = jnp.maximum(
                    acc1[...] + b0_ref[...], 0.0).astype(jnp.bfloat16)
                s12_ref[i] = jnp.dot(
                    hh, w12_ref[...], preferred_element_type=jnp.float32
                ).astype(jnp.bfloat16)

        @pl.when((t >= NM2) | (j < i))
        def _pass2_term():
            oacc[i] = oacc[i] + jnp.dot(
                at, s12_ref[j], preferred_element_type=jnp.float32)

        @pl.when((t >= NM2) & (j == NMC - 1))
        def _emit():
            out_ref[...] = (
                oacc[i]
                + jnp.dot(sl, s12e_ref[...],
                          preferred_element_type=jnp.float32)
                + b12_ref[...])

    @pl.when(i == NMC)
    def _edge_row():
        at = abuf[slot, pl.ds(0, RE)].astype(jnp.bfloat16)
        sl = sliv_ref[pl.ds(NMC * B, RE)]

        d = jnp.dot(at, s0_ref[j], preferred_element_type=jnp.float32)
        first = d + jnp.dot(sl, s0e_ref[...],
                            preferred_element_type=jnp.float32)
        prev = acc1[pl.ds(0, RE)]
        acc1[pl.ds(0, RE)] = jnp.where(j == 0, first, prev + d)

        oacc[i, pl.ds(0, RE)] = oacc[i, pl.ds(0, RE)] + jnp.dot(
            at, s12_ref[j], preferred_element_type=jnp.float32)

        @pl.when(j == NMC - 1)
        def _finalize_emit():
            hh = jnp.maximum(
                acc1[pl.ds(0, RE)] + b0_ref[...], 0.0).astype(jnp.bfloat16)
            s12e_ref[...] = jnp.dot(
                hh, w12_ref[...], preferred_element_type=jnp.float32
            ).astype(jnp.bfloat16)
            out_ref[pl.ds(0, RE), :] = (
                oacc[i, pl.ds(0, RE)]
                + jnp.dot(sl, s12e_ref[...],
                          preferred_element_type=jnp.float32)
                + b12_ref[...])


@jax.jit
def kernel(H, A, W0, b0, W1, b1, W2, b2):
    n, nfeat = H.shape
    nhid = W0.shape[1]
    nclass = W1.shape[1]
    ndim = W2.shape[1]

    s0 = pl.pallas_call(
        _s0_kernel,
        grid=(n // SB,),
        in_specs=[
            pl.BlockSpec((SB, nfeat), lambda i: (i, 0)),
            pl.BlockSpec((nfeat, nhid), lambda i: (0, 0)),
        ],
        out_specs=pl.BlockSpec((SB, nhid), lambda i: (i, 0)),
        out_shape=jax.ShapeDtypeStruct((n, nhid), jnp.bfloat16),
    )(H, W0)
    s0_3d = s0[:NMC * B].reshape(NMC, B, nhid)
    s0e = s0[NMC * B:]
    sliver = A[:, NMC * B:].astype(jnp.bfloat16)

    w12 = jnp.concatenate([W1, W2], axis=1).astype(jnp.bfloat16)
    b12 = jnp.concatenate([b1, b2])[None, :]
    b0_2d = b0[None, :]
    ncat = nclass + ndim

    steps = [(i, j) for i in range(NMR) for j in range(NMC)]
    steps += [(i, j) for i in range(NMC) for j in range(i, NMC)]
    i_tab = jnp.asarray(np.array([s[0] for s in steps], dtype=np.int32))
    j_tab = jnp.asarray(np.array([s[1] for s in steps], dtype=np.int32))

    full = lambda shape: pl.BlockSpec(shape, lambda t, it, jt: (0,) * len(shape))

    grid_spec = pltpu.PrefetchScalarGridSpec(
        num_scalar_prefetch=2,
        grid=(NT,),
        in_specs=[
            pl.BlockSpec(memory_space=pl.ANY),
            full((NMC, B, nhid)),
            full((RE, nhid)),
            full((n, RE)),
            full((1, nhid)),
            full((nhid, ncat)),
            full((1, ncat)),
        ],
        out_specs=pl.BlockSpec(
            (B, ncat),
            lambda t, it, jt: (jnp.where(t < NM2, NMC, it[t]), 0)),
        scratch_shapes=[
            pltpu.VMEM((2, B, B), jnp.float32),
            pltpu.VMEM((B, nhid), jnp.float32),
            pltpu.VMEM((NMC, B, ncat), jnp.bfloat16),
            pltpu.VMEM((RE, ncat), jnp.bfloat16),
            pltpu.VMEM((NMR, B, ncat), jnp.float32),
            pltpu.SemaphoreType.DMA((2,)),
        ],
    )
    out = pl.pallas_call(
        _tri_kernel,
        grid_spec=grid_spec,
        out_shape=jax.ShapeDtypeStruct((n, ncat), jnp.float32),
        compiler_params=pltpu.CompilerParams(
            dimension_semantics=("arbitrary",),
        ),
    )(i_tab, j_tab, A, s0_3d, s0e, sliver, b0_2d, w12, b12)

    return (out[:, :nclass], out[:, nclass:])


# speedup vs baseline: 1.1741x; 1.0698x over previous
import jax
import jax.numpy as jnp
import numpy as np
from jax import lax
from jax.experimental import pallas as pl
from jax.experimental.pallas import tpu as pltpu

N = 10000
B = 1664
NMC = 6
RE = N - NMC * B
NMR = NMC + 1
NM2 = NMR * NMC
NT = NM2 + NMC * (NMC + 1) // 2
SB = 2000


def _s0_kernel(h_ref, w0_ref, out_ref):
    out_ref[...] = jnp.dot(
        h_ref[...].astype(jnp.bfloat16),
        w0_ref[...].astype(jnp.bfloat16),
        preferred_element_type=jnp.float32,
    ).astype(jnp.bfloat16)


def _tri_kernel(i_tab, j_tab, a_hbm, s0_ref, s0e_ref, sliv_ref, b0_ref,
                w12_ref, b12_ref, out_ref, abuf, acc1, s12_ref, s12e_ref,
                oacc, sem):
    t = pl.program_id(0)
    i = i_tab[t]
    j = j_tab[t]
    slot = lax.rem(t, 3)

    def start_copy(u, s):
        iu = i_tab[u]
        ju = j_tab[u]
        r0 = iu * B
        c0 = ju * B

        @pl.when(iu < NMC)
        def _full():
            pltpu.make_async_copy(
                a_hbm.at[pl.ds(r0, B), pl.ds(c0, B)],
                abuf.at[s], sem.at[s]).start()

        @pl.when(iu == NMC)
        def _edge():
            pltpu.make_async_copy(
                a_hbm.at[pl.ds(r0, RE), pl.ds(c0, B)],
                abuf.at[s, pl.ds(0, RE)], sem.at[s]).start()

    def wait_copy(u, s):
        iu = i_tab[u]

        @pl.when(iu < NMC)
        def _full():
            pltpu.make_async_copy(
                a_hbm.at[pl.ds(0, B), pl.ds(0, B)],
                abuf.at[s], sem.at[s]).wait()

        @pl.when(iu == NMC)
        def _edge():
            pltpu.make_async_copy(
                a_hbm.at[pl.ds(0, RE), pl.ds(0, B)],
                abuf.at[s, pl.ds(0, RE)], sem.at[s]).wait()

    @pl.when(t == 0)
    def _prime():
        oacc[...] = jnp.zeros_like(oacc)
        start_copy(0, 0)
        start_copy(1, 1)

    @pl.when(t + 2 < NT)
    def _prefetch():
        start_copy(t + 2, lax.rem(t + 2, 3))

    wait_copy(t, slot)

    @pl.when(i < NMC)
    def _interior():
        at = abuf[slot].astype(jnp.bfloat16)
        sl = sliv_ref[pl.ds(i * B, B)]

        @pl.when(t < NM2)
        def _pass1():
            d = jnp.dot(at, s0_ref[j], preferred_element_type=jnp.float32)
            first = d + jnp.dot(sl, s0e_ref[...],
                                preferred_element_type=jnp.float32)
            acc1[...] = jnp.where(j == 0, first, acc1[...] + d)

            @pl.when(j == NMC - 1)
            def _finalize():
                hh = jnp.maximum(
                    acc1[...] + b0_ref[...], 0.0).astype(jnp.bfloat16)
                s12_ref[i] = jnp.dot(
                    hh, w12_ref[...], preferred_element_type=jnp.float32
                ).astype(jnp.bfloat16)

        @pl.when((t >= NM2) | (j < i))
        def _pass2_term():
            oacc[i] = oacc[i] + jnp.dot(
                at, s12_ref[j], preferred_element_type=jnp.float32)

        @pl.when((t >= NM2) & (j == NMC - 1))
        def _emit():
            out_ref[...] = (
                oacc[i]
                + jnp.dot(sl, s12e_ref[...],
                          preferred_element_type=jnp.float32)
                + b12_ref[...])

    @pl.when(i == NMC)
    def _edge_row():
        at = abuf[slot, pl.ds(0, RE)].astype(jnp.bfloat16)
        sl = sliv_ref[pl.ds(NMC * B, RE)]

        d = jnp.dot(at, s0_ref[j], preferred_element_type=jnp.float32)
        first = d + jnp.dot(sl, s0e_ref[...],
                            preferred_element_type=jnp.float32)
        prev = acc1[pl.ds(0, RE)]
        acc1[pl.ds(0, RE)] = jnp.where(j == 0, first, prev + d)

        oacc[i, pl.ds(0, RE)] = oacc[i, pl.ds(0, RE)] + jnp.dot(
            at, s12_ref[j], preferred_element_type=jnp.float32)

        @pl.when(j == NMC - 1)
        def _finalize_emit():
            hh = jnp.maximum(
                acc1[pl.ds(0, RE)] + b0_ref[...], 0.0).astype(jnp.bfloat16)
            s12e_ref[...] = jnp.dot(
                hh, w12_ref[...], preferred_element_type=jnp.float32
            ).astype(jnp.bfloat16)
            out_ref[pl.ds(0, RE), :] = (
                oacc[i, pl.ds(0, RE)]
                + jnp.dot(sl, s12e_ref[...],
                          preferred_element_type=jnp.float32)
                + b12_ref[...])


@jax.jit
def kernel(H, A, W0, b0, W1, b1, W2, b2):
    n, nfeat = H.shape
    nhid = W0.shape[1]
    nclass = W1.shape[1]
    ndim = W2.shape[1]

    s0 = pl.pallas_call(
        _s0_kernel,
        grid=(n // SB,),
        in_specs=[
            pl.BlockSpec((SB, nfeat), lambda i: (i, 0)),
            pl.BlockSpec((nfeat, nhid), lambda i: (0, 0)),
        ],
        out_specs=pl.BlockSpec((SB, nhid), lambda i: (i, 0)),
        out_shape=jax.ShapeDtypeStruct((n, nhid), jnp.bfloat16),
    )(H, W0)
    s0_3d = s0[:NMC * B].reshape(NMC, B, nhid)
    s0e = s0[NMC * B:]
    sliver = A[:, NMC * B:].astype(jnp.bfloat16)

    w12 = jnp.concatenate([W1, W2], axis=1).astype(jnp.bfloat16)
    b12 = jnp.concatenate([b1, b2])[None, :]
    b0_2d = b0[None, :]
    ncat = nclass + ndim

    steps = [(i, j) for i in range(NMR) for j in range(NMC)]
    steps += [(i, j) for i in range(NMC) for j in range(i, NMC)]
    i_tab = jnp.asarray(np.array([s[0] for s in steps], dtype=np.int32))
    j_tab = jnp.asarray(np.array([s[1] for s in steps], dtype=np.int32))

    full = lambda shape: pl.BlockSpec(shape, lambda t, it, jt: (0,) * len(shape))

    grid_spec = pltpu.PrefetchScalarGridSpec(
        num_scalar_prefetch=2,
        grid=(NT,),
        in_specs=[
            pl.BlockSpec(memory_space=pl.ANY),
            full((NMC, B, nhid)),
            full((RE, nhid)),
            full((n, RE)),
            full((1, nhid)),
            full((nhid, ncat)),
            full((1, ncat)),
        ],
        out_specs=pl.BlockSpec(
            (B, ncat),
            lambda t, it, jt: (jnp.where(t < NM2, NMC, it[t]), 0)),
        scratch_shapes=[
            pltpu.VMEM((2, B, B), jnp.float32),
            pltpu.VMEM((B, nhid), jnp.float32),
            pltpu.VMEM((NMC, B, ncat), jnp.bfloat16),
            pltpu.VMEM((RE, ncat), jnp.bfloat16),
            pltpu.VMEM((NMR, B, ncat), jnp.float32),
            pltpu.SemaphoreType.DMA((2,)),
        ],
    )
    out = pl.pallas_call(
        _tri_kernel,
        grid_spec=grid_spec,
        out_shape=jax.ShapeDtypeStruct((n, ncat), jnp.float32),
        compiler_params=pltpu.CompilerParams(
            dimension_semantics=("arbitrary",),
        ),
    )(i_tab, j_tab, A, s0_3d, s0e, sliver, b0_2d, w12, b12)

    return (out[:, :nclass], out[:, nclass:])
